# baseline (speedup 1.0000x reference)
import jax
import jax.numpy as jnp
from jax import lax
from jax.experimental import pallas as pl
from jax.experimental.pallas import tpu as pltpu
from jax.experimental.pallas import tpu_sc as plsc

BATCH = 16384
D = 128
NUM_LAYERS = 4

_NC = 2
_NS = 16
_NW = _NC * _NS
_CHUNK = 128

_NSPLIT = 4
_PART = BATCH // _NSPLIT


def _make_gather(batch, id_chunk_off):
    rows_per_w = batch // _NW
    nchunk = rows_per_w // _CHUNK
    ntot = 2 * nchunk
    nbuf = min(4, ntot)

    def body(uid_hbm, iid_hbm, utab_hbm, itab_hbm, u_out, i_out,
             idx_v, rows_v, *sems):
        gsem, wsem = sems[:nbuf], sems[nbuf:]
        wid = lax.axis_index("s") * _NC + lax.axis_index("c")
        base = wid * rows_per_w
        pltpu.sync_copy(uid_hbm.at[pl.ds(id_chunk_off + wid * nchunk, nchunk)],
                        idx_v.at[pl.ds(0, nchunk)])
        pltpu.sync_copy(iid_hbm.at[pl.ds(id_chunk_off + wid * nchunk, nchunk)],
                        idx_v.at[pl.ds(nchunk, nchunk)])

        def gather(k, b):
            tab = utab_hbm if k < nchunk else itab_hbm
            return pltpu.async_copy(tab.at[idx_v.at[k]], rows_v.at[b],
                                    gsem[b])

        def wback(k, b):
            out = u_out if k < nchunk else i_out
            off = base + (k % nchunk) * _CHUNK
            return pltpu.async_copy(rows_v.at[b],
                                    out.at[pl.ds(off, _CHUNK)], wsem[b])

        g = [None] * ntot
        w = [None] * ntot
        g[0] = gather(0, 0)
        g[1] = gather(1, 1)
        for k in range(ntot):
            nk = k + 2
            if nk < ntot:
                if nk >= nbuf:
                    w[nk - nbuf].wait()
                g[nk] = gather(nk, nk % nbuf)
            g[k].wait()
            w[k] = wback(k, k % nbuf)
        for k in range(max(0, ntot - nbuf), ntot):
            w[k].wait()

    mesh = plsc.VectorSubcoreMesh(core_axis_name="c", subcore_axis_name="s")
    return pl.kernel(
        body, mesh=mesh,
        out_type=[jax.ShapeDtypeStruct((batch, D), jnp.float32),
                  jax.ShapeDtypeStruct((batch, D), jnp.float32)],
        scratch_types=[
            pltpu.VMEM((ntot, _CHUNK), jnp.int32),
            pltpu.VMEM((nbuf, _CHUNK, D), jnp.float32),
        ] + [pltpu.SemaphoreType.DMA] * (2 * nbuf),
    )


def _sc_gather_part(uid2, iid2, user_table, item_table, part):
    fn = _make_gather(_PART, part * (_PART // _CHUNK))
    return fn(uid2, iid2, user_table, item_table)


_BB = 2048


def _mlp_body(u_ref, i_ref, wut_ref, bu_ref, wit_ref, bi_ref,
              r_ref, uo_ref, io_ref):
    wut = wut_ref[...]
    bu = bu_ref[...]
    u = u_ref[...]
    for _ in range(NUM_LAYERS):
        u = jnp.maximum(
            lax.dot_general(u, wut, (((1,), (0,)), ((), ())),
                            preferred_element_type=jnp.float32) + bu, 0.0)
    wit = wit_ref[...]
    bi = bi_ref[...]
    it = i_ref[...]
    for _ in range(NUM_LAYERS):
        it = jnp.maximum(
            lax.dot_general(it, wit, (((1,), (0,)), ((), ())),
                            preferred_element_type=jnp.float32) + bi, 0.0)
    uo_ref[...] = u
    io_ref[...] = it
    p = u * it
    ones_row = jnp.ones((1, D), jnp.float32)
    rows = []
    for g in range(_BB // D):
        pg = lax.slice(p, (g * D, 0), ((g + 1) * D, D))
        rows.append(lax.dot_general(ones_row, pg, (((1,), (1,)), ((), ())),
                                    preferred_element_type=jnp.float32))
    r_ref[...] = jnp.maximum(jnp.concatenate(rows, axis=0), 0.0)


def _mlp_body_aliased(u_ref, i_ref, wut_ref, bu_ref, wit_ref, bi_ref,
                      pr_ref, pu_ref, pi_ref, r_ref, uo_ref, io_ref):
    del pr_ref, pu_ref, pi_ref
    _mlp_body(u_ref, i_ref, wut_ref, bu_ref, wit_ref, bi_ref,
              r_ref, uo_ref, io_ref)


def _tc_mlp_part(u_rows, i_rows, WuT, bu, WiT, bi, part, prev=None):
    grid = _PART // _BB
    off = part * grid
    in_specs = [
        pl.BlockSpec((_BB, D), lambda b: (b, 0)),
        pl.BlockSpec((_BB, D), lambda b: (b, 0)),
        pl.BlockSpec((D, D), lambda b: (0, 0)),
        pl.BlockSpec((1, D), lambda b: (0, 0)),
        pl.BlockSpec((D, D), lambda b: (0, 0)),
        pl.BlockSpec((1, D), lambda b: (0, 0)),
    ]
    out_specs = [
        pl.BlockSpec((_BB // D, D), lambda b: (b + off, 0)),
        pl.BlockSpec((_BB, D), lambda b: (b + off, 0)),
        pl.BlockSpec((_BB, D), lambda b: (b + off, 0)),
    ]
    out_shape = [
        jax.ShapeDtypeStruct((BATCH // D, D), jnp.float32),
        jax.ShapeDtypeStruct((BATCH, D), jnp.float32),
        jax.ShapeDtypeStruct((BATCH, D), jnp.float32),
    ]
    args = [u_rows, i_rows, WuT, bu.reshape(1, D), WiT, bi.reshape(1, D)]
    if prev is None:
        body = _mlp_body
        aliases = {}
    else:
        body = _mlp_body_aliased
        in_specs = in_specs + [pl.BlockSpec(memory_space=pl.ANY)] * 3
        args = args + list(prev)
        aliases = {6: 0, 7: 1, 8: 2}
    return pl.pallas_call(
        body,
        grid=(grid,),
        in_specs=in_specs,
        out_specs=out_specs,
        out_shape=out_shape,
        input_output_aliases=aliases,
    )(*args)


def kernel(user_ids, item_ids, user_table, item_table, Wu, bu, Wi, bi):
    WuT, WiT = Wu.T, Wi.T
    uid2 = user_ids.reshape(BATCH // _CHUNK, _CHUNK)
    iid2 = item_ids.reshape(BATCH // _CHUNK, _CHUNK)
    gathered = [_sc_gather_part(uid2, iid2, user_table, item_table, q)
                for q in range(_NSPLIT)]
    outs = None
    for q in range(_NSPLIT):
        uq, iq = gathered[q]
        outs = _tc_mlp_part(uq, iq, WuT, bu, WiT, bi, part=q, prev=outs)
    r2, users_latent, items_latent = outs
    return (r2.reshape(BATCH), users_latent, items_latent)


# speedup vs baseline: 1.1333x; 1.1333x over previous
import jax
import jax.numpy as jnp
from jax import lax
from jax.experimental import pallas as pl
from jax.experimental.pallas import tpu as pltpu
from jax.experimental.pallas import tpu_sc as plsc

BATCH = 16384
D = 128
NUM_LAYERS = 4

_NC = 2
_NS = 16
_NW = _NC * _NS
_CHUNK = 128

_NSPLIT = 2
_PART = BATCH // _NSPLIT


def _make_gather(batch, id_chunk_off):
    rows_per_w = batch // _NW
    nchunk = rows_per_w // _CHUNK
    ntot = 2 * nchunk
    nbuf = min(4, ntot)

    def body(uid_hbm, iid_hbm, utab_hbm, itab_hbm, u_out, i_out,
             idx_v, rows_v, *sems):
        gsem, wsem = sems[:nbuf], sems[nbuf:]
        wid = lax.axis_index("s") * _NC + lax.axis_index("c")
        base = wid * rows_per_w
        pltpu.sync_copy(uid_hbm.at[pl.ds(id_chunk_off + wid * nchunk, nchunk)],
                        idx_v.at[pl.ds(0, nchunk)])
        pltpu.sync_copy(iid_hbm.at[pl.ds(id_chunk_off + wid * nchunk, nchunk)],
                        idx_v.at[pl.ds(nchunk, nchunk)])

        def gather(k, b):
            tab = utab_hbm if k < nchunk else itab_hbm
            return pltpu.async_copy(tab.at[idx_v.at[k]], rows_v.at[b],
                                    gsem[b])

        def wback(k, b):
            out = u_out if k < nchunk else i_out
            off = base + (k % nchunk) * _CHUNK
            return pltpu.async_copy(rows_v.at[b],
                                    out.at[pl.ds(off, _CHUNK)], wsem[b])

        g = [None] * ntot
        w = [None] * ntot
        g[0] = gather(0, 0)
        g[1] = gather(1, 1)
        for k in range(ntot):
            nk = k + 2
            if nk < ntot:
                if nk >= nbuf:
                    w[nk - nbuf].wait()
                g[nk] = gather(nk, nk % nbuf)
            g[k].wait()
            w[k] = wback(k, k % nbuf)
        for k in range(max(0, ntot - nbuf), ntot):
            w[k].wait()

    mesh = plsc.VectorSubcoreMesh(core_axis_name="c", subcore_axis_name="s")
    return pl.kernel(
        body, mesh=mesh,
        out_type=[jax.ShapeDtypeStruct((batch, D), jnp.float32),
                  jax.ShapeDtypeStruct((batch, D), jnp.float32)],
        scratch_types=[
            pltpu.VMEM((ntot, _CHUNK), jnp.int32),
            pltpu.VMEM((nbuf, _CHUNK, D), jnp.float32),
        ] + [pltpu.SemaphoreType.DMA] * (2 * nbuf),
    )


def _sc_gather_part(uid2, iid2, user_table, item_table, part):
    fn = _make_gather(_PART, part * (_PART // _CHUNK))
    return fn(uid2, iid2, user_table, item_table)


_BB = 2048


def _mlp_body(u_ref, i_ref, wut_ref, bu_ref, wit_ref, bi_ref,
              r_ref, uo_ref, io_ref):
    wut = wut_ref[...]
    bu = bu_ref[...]
    u = u_ref[...]
    for _ in range(NUM_LAYERS):
        u = jnp.maximum(
            lax.dot_general(u, wut, (((1,), (0,)), ((), ())),
                            preferred_element_type=jnp.float32) + bu, 0.0)
    wit = wit_ref[...]
    bi = bi_ref[...]
    it = i_ref[...]
    for _ in range(NUM_LAYERS):
        it = jnp.maximum(
            lax.dot_general(it, wit, (((1,), (0,)), ((), ())),
                            preferred_element_type=jnp.float32) + bi, 0.0)
    uo_ref[...] = u
    io_ref[...] = it
    p = u * it
    ones_row = jnp.ones((1, D), jnp.float32)
    rows = []
    for g in range(_BB // D):
        pg = lax.slice(p, (g * D, 0), ((g + 1) * D, D))
        rows.append(lax.dot_general(ones_row, pg, (((1,), (1,)), ((), ())),
                                    preferred_element_type=jnp.float32))
    r_ref[...] = jnp.maximum(jnp.concatenate(rows, axis=0), 0.0)


def _mlp_body_aliased(u_ref, i_ref, wut_ref, bu_ref, wit_ref, bi_ref,
                      pr_ref, pu_ref, pi_ref, r_ref, uo_ref, io_ref):
    del pr_ref, pu_ref, pi_ref
    _mlp_body(u_ref, i_ref, wut_ref, bu_ref, wit_ref, bi_ref,
              r_ref, uo_ref, io_ref)


def _tc_mlp_part(u_rows, i_rows, WuT, bu, WiT, bi, part, prev=None):
    grid = _PART // _BB
    off = part * grid
    in_specs = [
        pl.BlockSpec((_BB, D), lambda b: (b, 0)),
        pl.BlockSpec((_BB, D), lambda b: (b, 0)),
        pl.BlockSpec((D, D), lambda b: (0, 0)),
        pl.BlockSpec((1, D), lambda b: (0, 0)),
        pl.BlockSpec((D, D), lambda b: (0, 0)),
        pl.BlockSpec((1, D), lambda b: (0, 0)),
    ]
    out_specs = [
        pl.BlockSpec((_BB // D, D), lambda b: (b + off, 0)),
        pl.BlockSpec((_BB, D), lambda b: (b + off, 0)),
        pl.BlockSpec((_BB, D), lambda b: (b + off, 0)),
    ]
    out_shape = [
        jax.ShapeDtypeStruct((BATCH // D, D), jnp.float32),
        jax.ShapeDtypeStruct((BATCH, D), jnp.float32),
        jax.ShapeDtypeStruct((BATCH, D), jnp.float32),
    ]
    args = [u_rows, i_rows, WuT, bu.reshape(1, D), WiT, bi.reshape(1, D)]
    if prev is None:
        body = _mlp_body
        aliases = {}
    else:
        body = _mlp_body_aliased
        in_specs = in_specs + [pl.BlockSpec(memory_space=pl.ANY)] * 3
        args = args + list(prev)
        aliases = {6: 0, 7: 1, 8: 2}
    return pl.pallas_call(
        body,
        grid=(grid,),
        in_specs=in_specs,
        out_specs=out_specs,
        out_shape=out_shape,
        input_output_aliases=aliases,
    )(*args)


def kernel(user_ids, item_ids, user_table, item_table, Wu, bu, Wi, bi):
    WuT, WiT = Wu.T, Wi.T
    uid2 = user_ids.reshape(BATCH // _CHUNK, _CHUNK)
    iid2 = item_ids.reshape(BATCH // _CHUNK, _CHUNK)
    gathered = [_sc_gather_part(uid2, iid2, user_table, item_table, q)
                for q in range(_NSPLIT)]
    outs = None
    for q in range(_NSPLIT):
        uq, iq = gathered[q]
        outs = _tc_mlp_part(uq, iq, WuT, bu, WiT, bi, part=q, prev=outs)
    r2, users_latent, items_latent = outs
    return (r2.reshape(BATCH), users_latent, items_latent)


# speedup vs baseline: 1.1503x; 1.0150x over previous
import jax
import jax.numpy as jnp
from jax import lax
from jax.experimental import pallas as pl
from jax.experimental.pallas import tpu as pltpu
from jax.experimental.pallas import tpu_sc as plsc

BATCH = 16384
D = 128
NUM_LAYERS = 4

_NC = 2
_NS = 16
_NW = _NC * _NS
_CHUNK = 128

_NSPLIT = 2
_PART = BATCH // _NSPLIT


def _make_gather(batch, id_chunk_off):
    rows_per_w = batch // _NW
    nchunk = rows_per_w // _CHUNK
    ntot = 2 * nchunk
    nbuf = min(4, ntot)

    inflight = min(nbuf, ntot)

    def body(uid_hbm, iid_hbm, utab_hbm, itab_hbm, u_out, i_out,
             idx_v, rows_v, *sems):
        gsem, wsem = sems[:nbuf], sems[nbuf:]
        isem = sems[2 * nbuf]
        wid = lax.axis_index("s") * _NC + lax.axis_index("c")
        base = wid * rows_per_w
        c1 = pltpu.async_copy(
            uid_hbm.at[pl.ds(id_chunk_off + wid * nchunk, nchunk)],
            idx_v.at[pl.ds(0, nchunk)], isem)
        c2 = pltpu.async_copy(
            iid_hbm.at[pl.ds(id_chunk_off + wid * nchunk, nchunk)],
            idx_v.at[pl.ds(nchunk, nchunk)], isem)
        c1.wait()
        c2.wait()

        def gather(k, b):
            tab = utab_hbm if k < nchunk else itab_hbm
            return pltpu.async_copy(tab.at[idx_v.at[k]], rows_v.at[b],
                                    gsem[b])

        def wback(k, b):
            out = u_out if k < nchunk else i_out
            off = base + (k % nchunk) * _CHUNK
            return pltpu.async_copy(rows_v.at[b],
                                    out.at[pl.ds(off, _CHUNK)], wsem[b])

        g = [None] * ntot
        w = [None] * ntot
        for k in range(inflight):
            g[k] = gather(k, k % nbuf)
        for k in range(ntot):
            nk = k + inflight
            if nk < ntot:
                if nk >= nbuf:
                    w[nk - nbuf].wait()
                g[nk] = gather(nk, nk % nbuf)
            g[k].wait()
            w[k] = wback(k, k % nbuf)
        for k in range(max(0, ntot - nbuf), ntot):
            w[k].wait()

    mesh = plsc.VectorSubcoreMesh(core_axis_name="c", subcore_axis_name="s")
    return pl.kernel(
        body, mesh=mesh,
        out_type=[jax.ShapeDtypeStruct((batch, D), jnp.float32),
                  jax.ShapeDtypeStruct((batch, D), jnp.float32)],
        scratch_types=[
            pltpu.VMEM((ntot, _CHUNK), jnp.int32),
            pltpu.VMEM((nbuf, _CHUNK, D), jnp.float32),
        ] + [pltpu.SemaphoreType.DMA] * (2 * nbuf + 1),
    )


def _sc_gather_part(uid2, iid2, user_table, item_table, part):
    fn = _make_gather(_PART, part * (_PART // _CHUNK))
    return fn(uid2, iid2, user_table, item_table)


_BB = 2048


def _mlp_body(u_ref, i_ref, wut_ref, bu_ref, wit_ref, bi_ref,
              r_ref, uo_ref, io_ref):
    wut = wut_ref[...]
    bu = bu_ref[...]
    u = u_ref[...]
    for _ in range(NUM_LAYERS):
        u = jnp.maximum(
            lax.dot_general(u, wut, (((1,), (0,)), ((), ())),
                            preferred_element_type=jnp.float32) + bu, 0.0)
    wit = wit_ref[...]
    bi = bi_ref[...]
    it = i_ref[...]
    for _ in range(NUM_LAYERS):
        it = jnp.maximum(
            lax.dot_general(it, wit, (((1,), (0,)), ((), ())),
                            preferred_element_type=jnp.float32) + bi, 0.0)
    uo_ref[...] = u
    io_ref[...] = it
    p = u * it
    ones_row = jnp.ones((1, D), jnp.float32)
    rows = []
    for g in range(_BB // D):
        pg = lax.slice(p, (g * D, 0), ((g + 1) * D, D))
        rows.append(lax.dot_general(ones_row, pg, (((1,), (1,)), ((), ())),
                                    preferred_element_type=jnp.float32))
    r_ref[...] = jnp.maximum(jnp.concatenate(rows, axis=0), 0.0)


def _mlp_body_aliased(u_ref, i_ref, wut_ref, bu_ref, wit_ref, bi_ref,
                      pr_ref, pu_ref, pi_ref, r_ref, uo_ref, io_ref):
    del pr_ref, pu_ref, pi_ref
    _mlp_body(u_ref, i_ref, wut_ref, bu_ref, wit_ref, bi_ref,
              r_ref, uo_ref, io_ref)


def _tc_mlp_part(u_rows, i_rows, WuT, bu, WiT, bi, part, prev=None):
    grid = _PART // _BB
    off = part * grid
    in_specs = [
        pl.BlockSpec((_BB, D), lambda b: (b, 0)),
        pl.BlockSpec((_BB, D), lambda b: (b, 0)),
        pl.BlockSpec((D, D), lambda b: (0, 0)),
        pl.BlockSpec((1, D), lambda b: (0, 0)),
        pl.BlockSpec((D, D), lambda b: (0, 0)),
        pl.BlockSpec((1, D), lambda b: (0, 0)),
    ]
    out_specs = [
        pl.BlockSpec((_BB // D, D), lambda b: (b + off, 0)),
        pl.BlockSpec((_BB, D), lambda b: (b + off, 0)),
        pl.BlockSpec((_BB, D), lambda b: (b + off, 0)),
    ]
    out_shape = [
        jax.ShapeDtypeStruct((BATCH // D, D), jnp.float32),
        jax.ShapeDtypeStruct((BATCH, D), jnp.float32),
        jax.ShapeDtypeStruct((BATCH, D), jnp.float32),
    ]
    args = [u_rows, i_rows, WuT, bu.reshape(1, D), WiT, bi.reshape(1, D)]
    if prev is None:
        body = _mlp_body
        aliases = {}
    else:
        body = _mlp_body_aliased
        in_specs = in_specs + [pl.BlockSpec(memory_space=pl.ANY)] * 3
        args = args + list(prev)
        aliases = {6: 0, 7: 1, 8: 2}
    return pl.pallas_call(
        body,
        grid=(grid,),
        in_specs=in_specs,
        out_specs=out_specs,
        out_shape=out_shape,
        input_output_aliases=aliases,
    )(*args)


def kernel(user_ids, item_ids, user_table, item_table, Wu, bu, Wi, bi):
    WuT, WiT = Wu.T, Wi.T
    uid2 = user_ids.reshape(BATCH // _CHUNK, _CHUNK)
    iid2 = item_ids.reshape(BATCH // _CHUNK, _CHUNK)
    gathered = [_sc_gather_part(uid2, iid2, user_table, item_table, q)
                for q in range(_NSPLIT)]
    outs = None
    for q in range(_NSPLIT):
        uq, iq = gathered[q]
        outs = _tc_mlp_part(uq, iq, WuT, bu, WiT, bi, part=q, prev=outs)
    r2, users_latent, items_latent = outs
    return (r2.reshape(BATCH), users_latent, items_latent)


# speedup vs baseline: 1.1693x; 1.0165x over previous
import jax
import jax.numpy as jnp
from jax import lax
from jax.experimental import pallas as pl
from jax.experimental.pallas import tpu as pltpu
from jax.experimental.pallas import tpu_sc as plsc

BATCH = 16384
D = 128
NUM_LAYERS = 4

_NC = 2
_NS = 16
_NW = _NC * _NS
_CHUNK = 128

_NSPLIT = 2
_PART = BATCH // _NSPLIT


def _make_gather(batch, id_chunk_off):
    rows_per_w = batch // _NW
    nchunk = rows_per_w // _CHUNK
    ntot = 2 * nchunk
    nbuf = min(4, ntot)

    inflight = min(nbuf, ntot)

    def body(uid_hbm, iid_hbm, utab_hbm, itab_hbm, u_out, i_out,
             idx_v, rows_v, *sems):
        gsem, wsem = sems[:nbuf], sems[nbuf:]
        isem = sems[2 * nbuf]
        wid = lax.axis_index("s") * _NC + lax.axis_index("c")
        base = wid * rows_per_w
        c1 = pltpu.async_copy(
            uid_hbm.at[pl.ds(id_chunk_off + wid * nchunk, nchunk)],
            idx_v.at[pl.ds(0, nchunk)], isem)
        c2 = pltpu.async_copy(
            iid_hbm.at[pl.ds(id_chunk_off + wid * nchunk, nchunk)],
            idx_v.at[pl.ds(nchunk, nchunk)], isem)
        c1.wait()
        c2.wait()

        def gather(k, b):
            tab = utab_hbm if k < nchunk else itab_hbm
            return pltpu.async_copy(tab.at[idx_v.at[k]], rows_v.at[b],
                                    gsem[b])

        def wback(k, b):
            out = u_out if k < nchunk else i_out
            off = base + (k % nchunk) * _CHUNK
            return pltpu.async_copy(rows_v.at[b],
                                    out.at[pl.ds(off, _CHUNK)], wsem[b])

        g = [None] * ntot
        w = [None] * ntot
        for k in range(inflight):
            g[k] = gather(k, k % nbuf)
        for k in range(ntot):
            nk = k + inflight
            if nk < ntot:
                if nk >= nbuf:
                    w[nk - nbuf].wait()
                g[nk] = gather(nk, nk % nbuf)
            g[k].wait()
            w[k] = wback(k, k % nbuf)
        for k in range(max(0, ntot - nbuf), ntot):
            w[k].wait()

    mesh = plsc.VectorSubcoreMesh(core_axis_name="c", subcore_axis_name="s")
    return pl.kernel(
        body, mesh=mesh,
        out_type=[jax.ShapeDtypeStruct((batch, D), jnp.float32),
                  jax.ShapeDtypeStruct((batch, D), jnp.float32)],
        scratch_types=[
            pltpu.VMEM((ntot, _CHUNK), jnp.int32),
            pltpu.VMEM((nbuf, _CHUNK, D), jnp.float32),
        ] + [pltpu.SemaphoreType.DMA] * (2 * nbuf + 1),
    )


def _sc_gather_part(uid2, iid2, user_table, item_table, part):
    fn = _make_gather(_PART, part * (_PART // _CHUNK))
    return fn(uid2, iid2, user_table, item_table)


_BB = 4096


def _mlp_body(u_ref, i_ref, wut_ref, bu_ref, wit_ref, bi_ref,
              r_ref, uo_ref, io_ref):
    wut = wut_ref[...]
    bu = bu_ref[...]
    u = u_ref[...]
    for _ in range(NUM_LAYERS):
        u = jnp.maximum(
            lax.dot_general(u, wut, (((1,), (0,)), ((), ())),
                            preferred_element_type=jnp.float32) + bu, 0.0)
    wit = wit_ref[...]
    bi = bi_ref[...]
    it = i_ref[...]
    for _ in range(NUM_LAYERS):
        it = jnp.maximum(
            lax.dot_general(it, wit, (((1,), (0,)), ((), ())),
                            preferred_element_type=jnp.float32) + bi, 0.0)
    uo_ref[...] = u
    io_ref[...] = it
    p = u * it
    ones_row = jnp.ones((1, D), jnp.float32)
    rows = []
    for g in range(_BB // D):
        pg = lax.slice(p, (g * D, 0), ((g + 1) * D, D))
        rows.append(lax.dot_general(ones_row, pg, (((1,), (1,)), ((), ())),
                                    preferred_element_type=jnp.float32))
    r_ref[...] = jnp.maximum(jnp.concatenate(rows, axis=0), 0.0)


def _mlp_body_aliased(u_ref, i_ref, wut_ref, bu_ref, wit_ref, bi_ref,
                      pr_ref, pu_ref, pi_ref, r_ref, uo_ref, io_ref):
    del pr_ref, pu_ref, pi_ref
    _mlp_body(u_ref, i_ref, wut_ref, bu_ref, wit_ref, bi_ref,
              r_ref, uo_ref, io_ref)


def _tc_mlp_part(u_rows, i_rows, WuT, bu, WiT, bi, part, prev=None):
    grid = _PART // _BB
    off = part * grid
    in_specs = [
        pl.BlockSpec((_BB, D), lambda b: (b, 0)),
        pl.BlockSpec((_BB, D), lambda b: (b, 0)),
        pl.BlockSpec((D, D), lambda b: (0, 0)),
        pl.BlockSpec((1, D), lambda b: (0, 0)),
        pl.BlockSpec((D, D), lambda b: (0, 0)),
        pl.BlockSpec((1, D), lambda b: (0, 0)),
    ]
    out_specs = [
        pl.BlockSpec((_BB // D, D), lambda b: (b + off, 0)),
        pl.BlockSpec((_BB, D), lambda b: (b + off, 0)),
        pl.BlockSpec((_BB, D), lambda b: (b + off, 0)),
    ]
    out_shape = [
        jax.ShapeDtypeStruct((BATCH // D, D), jnp.float32),
        jax.ShapeDtypeStruct((BATCH, D), jnp.float32),
        jax.ShapeDtypeStruct((BATCH, D), jnp.float32),
    ]
    args = [u_rows, i_rows, WuT, bu.reshape(1, D), WiT, bi.reshape(1, D)]
    if prev is None:
        body = _mlp_body
        aliases = {}
    else:
        body = _mlp_body_aliased
        in_specs = in_specs + [pl.BlockSpec(memory_space=pl.ANY)] * 3
        args = args + list(prev)
        aliases = {6: 0, 7: 1, 8: 2}
    return pl.pallas_call(
        body,
        grid=(grid,),
        in_specs=in_specs,
        out_specs=out_specs,
        out_shape=out_shape,
        input_output_aliases=aliases,
    )(*args)


def kernel(user_ids, item_ids, user_table, item_table, Wu, bu, Wi, bi):
    WuT, WiT = Wu.T, Wi.T
    uid2 = user_ids.reshape(BATCH // _CHUNK, _CHUNK)
    iid2 = item_ids.reshape(BATCH // _CHUNK, _CHUNK)
    gathered = [_sc_gather_part(uid2, iid2, user_table, item_table, q)
                for q in range(_NSPLIT)]
    outs = None
    for q in range(_NSPLIT):
        uq, iq = gathered[q]
        outs = _tc_mlp_part(uq, iq, WuT, bu, WiT, bi, part=q, prev=outs)
    r2, users_latent, items_latent = outs
    return (r2.reshape(BATCH), users_latent, items_latent)
